# Initial kernel scaffold; baseline (speedup 1.0000x reference)
import jax
import jax.numpy as jnp
from jax.experimental import pallas as pl


def kernel(inputs, embed, ln1_s, ln1_b, Wqk, Wv, Wo, ln2_s, ln2_b, W1, b1, W2, b2, lnf_s, lnf_b):
    raise NotImplementedError("write your pallas kernel here")


# speedup vs baseline: 3.2708x; 3.2708x over previous
import functools

import numpy as np
import jax
import jax.numpy as jnp
from jax import lax
from jax.experimental import pallas as pl
from jax.experimental.pallas import tpu as pltpu

B, S = 2, 2048
VOCAB, EMB, H, L = 32000, 1024, 16, 2
QKV, MLP = 1024, 4096
DH = QKV // H
N_HASHES, N_BUCKETS, CHUNK = 2, 32, 64
NCH = S // CHUNK
BH = B * H
R = BH * N_HASHES

BS = 512
QB = 512
KB = QB + CHUNK


def _sinusoidal_pe():
    pos = np.arange(S)[:, None].astype(np.float32)
    div = np.exp(np.arange(0, EMB, 2).astype(np.float32) * (-np.log(10000.0) / EMB))
    pe = np.zeros((S, EMB), np.float32)
    pe[:, 0::2] = np.sin(pos * div)
    pe[:, 1::2] = np.cos(pos * div)
    return jnp.asarray(pe)


def _ln(x, s, b):
    m = jnp.mean(x, axis=-1, keepdims=True)
    v = jnp.mean(jnp.square(x - m), axis=-1, keepdims=True)
    return (x - m) * lax.rsqrt(v + 1e-6) * s + b


def _qkv_body(x_ref, s_ref, b_ref, wqk_ref, wv_ref, qk_ref, v_ref):
    h = _ln(x_ref[...], s_ref[...], b_ref[...])
    qk_ref[...] = jnp.dot(h, wqk_ref[...], preferred_element_type=jnp.float32)
    v_ref[...] = jnp.dot(h, wv_ref[...], preferred_element_type=jnp.float32)


def _qkv(x, s, b, wqk, wv):
    n = (B * S) // BS
    return pl.pallas_call(
        _qkv_body,
        grid=(n,),
        in_specs=[
            pl.BlockSpec((BS, EMB), lambda i: (i, 0)),
            pl.BlockSpec((1, EMB), lambda i: (0, 0)),
            pl.BlockSpec((1, EMB), lambda i: (0, 0)),
            pl.BlockSpec((EMB, QKV), lambda i: (0, 0)),
            pl.BlockSpec((EMB, QKV), lambda i: (0, 0)),
        ],
        out_specs=[
            pl.BlockSpec((BS, QKV), lambda i: (i, 0)),
            pl.BlockSpec((BS, QKV), lambda i: (i, 0)),
        ],
        out_shape=[
            jax.ShapeDtypeStruct((B * S, QKV), jnp.float32),
            jax.ShapeDtypeStruct((B * S, QKV), jnp.float32),
        ],
    )(x, s.reshape(1, EMB), b.reshape(1, EMB), wqk, wv)


def _att_body(sqk_ref, sv_ref, sb_ref, sbt_ref, o_ref, lse_ref):
    qk = sqk_ref[0]
    vv = sv_ref[0]
    nrm = jnp.sqrt(jnp.sum(qk * qk, axis=-1, keepdims=True)) + 1e-6
    kn = qk / nrm
    k_ext = jnp.concatenate([kn[S - CHUNK:], kn], axis=0)
    v_ext = jnp.concatenate([vv[S - CHUNK:], vv], axis=0)
    b_row = sb_ref[0]
    b_ext = jnp.concatenate([b_row[:, S - CHUNK:], b_row], axis=1)
    bt = sbt_ref[0]

    qi = lax.broadcasted_iota(jnp.int32, (QB, KB), 0) // CHUNK
    kj = lax.broadcasted_iota(jnp.int32, (QB, KB), 1) // CHUNK
    band = (kj == qi) | (kj == qi + 1)

    for t in range(S // QB):
        p = t * QB
        q = qk[p:p + QB]
        kb = k_ext[p:p + KB]
        vb = v_ext[p:p + KB]
        dots = lax.dot_general(q, kb, (((1,), (1,)), ((), ())),
                               preferred_element_type=jnp.float32) * (1.0 / 8.0)
        mask = band & (bt[p:p + QB] == b_ext[:, p:p + KB])
        dots = jnp.where(mask, dots, -1e9)
        m = jnp.max(dots, axis=-1, keepdims=True)
        e = jnp.exp(dots - m)
        ssum = jnp.sum(e, axis=-1, keepdims=True)
        o_ref[0, p:p + QB, :] = jnp.dot(e, vb, preferred_element_type=jnp.float32) / ssum
        lse_ref[0, p:p + QB, :] = m + jnp.log(ssum)


def _attention(sqk, sv, sb):
    sb3 = sb.reshape(R, 1, S)
    sbt = sb.reshape(R, S, 1)
    return pl.pallas_call(
        _att_body,
        grid=(R,),
        in_specs=[
            pl.BlockSpec((1, S, DH), lambda i: (i, 0, 0)),
            pl.BlockSpec((1, S, DH), lambda i: (i, 0, 0)),
            pl.BlockSpec((1, 1, S), lambda i: (i, 0, 0)),
            pl.BlockSpec((1, S, 1), lambda i: (i, 0, 0)),
        ],
        out_specs=[
            pl.BlockSpec((1, S, DH), lambda i: (i, 0, 0)),
            pl.BlockSpec((1, S, 1), lambda i: (i, 0, 0)),
        ],
        out_shape=[
            jax.ShapeDtypeStruct((R, S, DH), jnp.float32),
            jax.ShapeDtypeStruct((R, S, 1), jnp.float32),
        ],
    )(sqk, sv, sb3, sbt)


def _ffn_body(x_ref, att_ref, wo_ref, s_ref, b_ref, w1_ref, b1_ref, w2_ref,
              b2_ref, y_ref):
    x1 = x_ref[...] + jnp.dot(att_ref[...], wo_ref[...],
                              preferred_element_type=jnp.float32)
    h2 = _ln(x1, s_ref[...], b_ref[...])
    g = jax.nn.gelu(jnp.dot(h2, w1_ref[...],
                            preferred_element_type=jnp.float32) + b1_ref[...])
    y_ref[...] = x1 + jnp.dot(g, w2_ref[...],
                              preferred_element_type=jnp.float32) + b2_ref[...]


def _ffn(x, att, wo, s, b, w1, b1, w2, b2):
    FBS = 256
    n = (B * S) // FBS
    return pl.pallas_call(
        _ffn_body,
        grid=(n,),
        in_specs=[
            pl.BlockSpec((FBS, EMB), lambda i: (i, 0)),
            pl.BlockSpec((FBS, QKV), lambda i: (i, 0)),
            pl.BlockSpec((QKV, EMB), lambda i: (0, 0)),
            pl.BlockSpec((1, EMB), lambda i: (0, 0)),
            pl.BlockSpec((1, EMB), lambda i: (0, 0)),
            pl.BlockSpec((EMB, MLP), lambda i: (0, 0)),
            pl.BlockSpec((1, MLP), lambda i: (0, 0)),
            pl.BlockSpec((MLP, EMB), lambda i: (0, 0)),
            pl.BlockSpec((1, EMB), lambda i: (0, 0)),
        ],
        out_specs=pl.BlockSpec((FBS, EMB), lambda i: (i, 0)),
        out_shape=jax.ShapeDtypeStruct((B * S, EMB), jnp.float32),
    )(x, att, wo, s.reshape(1, EMB), b.reshape(1, EMB), w1,
      b1.reshape(1, MLP), w2, b2.reshape(1, EMB))


def _lnf_body(x_ref, s_ref, b_ref, y_ref):
    y_ref[...] = _ln(x_ref[...], s_ref[...], b_ref[...])


def _lnf(x, s, b):
    n = (B * S) // BS
    return pl.pallas_call(
        _lnf_body,
        grid=(n,),
        in_specs=[
            pl.BlockSpec((BS, EMB), lambda i: (i, 0)),
            pl.BlockSpec((1, EMB), lambda i: (0, 0)),
            pl.BlockSpec((1, EMB), lambda i: (0, 0)),
        ],
        out_specs=pl.BlockSpec((BS, EMB), lambda i: (i, 0)),
        out_shape=jax.ShapeDtypeStruct((B * S, EMB), jnp.float32),
    )(x, s.reshape(1, EMB), b.reshape(1, EMB))


def kernel(inputs, embed, ln1_s, ln1_b, Wqk, Wv, Wo, ln2_s, ln2_b, W1, b1,
           W2, b2, lnf_s, lnf_b):
    pe = _sinusoidal_pe()
    x = jnp.take(embed, inputs.astype(jnp.int32), axis=0).reshape(B * S, EMB)
    x = x + jnp.tile(pe, (B, 1))

    for l in range(L):
        qk_f, v_f = _qkv(x, ln1_s[l], ln1_b[l], Wqk[l], Wv[l])

        qk = qk_f.reshape(B, S, H, DH).transpose(0, 2, 1, 3).reshape(BH, S, DH)
        vv = v_f.reshape(B, S, H, DH).transpose(0, 2, 1, 3).reshape(BH, S, DH)

        rot = jax.random.normal(jax.random.fold_in(jax.random.key(42), l),
                                (DH, N_HASHES, N_BUCKETS // 2), jnp.float32)
        rotated = jnp.einsum('bsd,dhr->bhsr', qk, rot)
        buckets = jnp.argmax(jnp.concatenate([rotated, -rotated], axis=-1),
                             axis=-1).reshape(R, S)

        ticker = jnp.argsort(buckets * S + jnp.arange(S, dtype=jnp.int32)[None, :],
                             axis=-1)
        undo = jnp.argsort(ticker, axis=-1)

        qk_e = jnp.broadcast_to(qk[:, None], (BH, N_HASHES, S, DH)).reshape(R, S, DH)
        v_e = jnp.broadcast_to(vv[:, None], (BH, N_HASHES, S, DH)).reshape(R, S, DH)
        sqk = jnp.take_along_axis(qk_e, ticker[..., None], axis=1)
        sv = jnp.take_along_axis(v_e, ticker[..., None], axis=1)
        sb = jnp.take_along_axis(buckets, ticker, axis=1).astype(jnp.float32)

        o_s, lse_s = _attention(sqk, sv, sb)

        o = jnp.take_along_axis(o_s, undo[..., None], axis=1)
        lse = jnp.take_along_axis(lse_s[..., 0], undo, axis=1)
        o = o.reshape(BH, N_HASHES, S, DH)
        lse = lse.reshape(BH, N_HASHES, S)
        w = jax.nn.softmax(lse, axis=1)
        att = jnp.sum(o * w[..., None], axis=1)
        att = att.reshape(B, H, S, DH).transpose(0, 2, 1, 3).reshape(B * S, QKV)

        x = _ffn(x, att, Wo[l], ln2_s[l], ln2_b[l], W1[l], b1[l], W2[l], b2[l])

    return _lnf(x, lnf_s, lnf_b).reshape(B, S, EMB)


# speedup vs baseline: 4.1620x; 1.2725x over previous
import functools

import numpy as np
import jax
import jax.numpy as jnp
from jax import lax
from jax.experimental import pallas as pl
from jax.experimental.pallas import tpu as pltpu
from jax.experimental.pallas import tpu_sc as plsc

B, S = 2, 2048
VOCAB, EMB, H, L = 32000, 1024, 16, 2
QKV, MLP = 1024, 4096
DH = QKV // H
N_HASHES, N_BUCKETS, CHUNK = 2, 32, 64
NCH = S // CHUNK
BH = B * H
R = BH * N_HASHES

BS = 512
QB = 512
KB = QB + CHUNK


def _sinusoidal_pe():
    pos = np.arange(S)[:, None].astype(np.float32)
    div = np.exp(np.arange(0, EMB, 2).astype(np.float32) * (-np.log(10000.0) / EMB))
    pe = np.zeros((S, EMB), np.float32)
    pe[:, 0::2] = np.sin(pos * div)
    pe[:, 1::2] = np.cos(pos * div)
    return jnp.asarray(pe)


def _ln(x, s, b):
    m = jnp.mean(x, axis=-1, keepdims=True)
    v = jnp.mean(jnp.square(x - m), axis=-1, keepdims=True)
    return (x - m) * lax.rsqrt(v + 1e-6) * s + b


def _qkv_body(x_ref, s_ref, b_ref, w_ref, qkv_ref):
    h = _ln(x_ref[...], s_ref[...], b_ref[...])
    qkv_ref[...] = jnp.dot(h, w_ref[...], preferred_element_type=jnp.float32)


def _qkv(x, s, b, w_int):
    n = (B * S) // BS
    return pl.pallas_call(
        _qkv_body,
        grid=(n,),
        in_specs=[
            pl.BlockSpec((BS, EMB), lambda i: (i, 0)),
            pl.BlockSpec((1, EMB), lambda i: (0, 0)),
            pl.BlockSpec((1, EMB), lambda i: (0, 0)),
            pl.BlockSpec((EMB, 2 * QKV), lambda i: (0, 0)),
        ],
        out_specs=pl.BlockSpec((BS, 2 * QKV), lambda i: (i, 0)),
        out_shape=jax.ShapeDtypeStruct((B * S, 2 * QKV), jnp.float32),
    )(x, s.reshape(1, EMB), b.reshape(1, EMB), w_int)


CSB = 256


def _rank_body(qk_ref, r2_ref, d0_ref, d1_ref, off0_ref, off1_ref):
    i = pl.program_id(0)

    lane = lax.broadcasted_iota(jnp.int32, (S, N_BUCKETS), 1)
    tr = (lax.broadcasted_iota(jnp.int32, (CSB, CSB), 0)
          >= lax.broadcasted_iota(jnp.int32, (CSB, CSB), 1)).astype(jnp.float32)
    up = (lax.broadcasted_iota(jnp.int32, (N_BUCKETS, N_BUCKETS), 0)
          < lax.broadcasted_iota(jnp.int32, (N_BUCKETS, N_BUCKETS), 1)
          ).astype(jnp.float32)

    bh = i
    qk = qk_ref[:, 0:DH]
    scores = jnp.dot(qk, r2_ref[...], preferred_element_type=jnp.float32)
    for n in range(N_HASHES):
        sn = scores[:, n * N_BUCKETS:(n + 1) * N_BUCKETS]
        m = jnp.max(sn, axis=-1, keepdims=True)
        bidx = jnp.min(jnp.where(sn == m, lane, N_BUCKETS), axis=-1,
                       keepdims=True)
        onehot = (lane == bidx).astype(jnp.float32)
        carry = jnp.zeros((1, N_BUCKETS), jnp.float32)
        parts = []
        for k in range(S // CSB):
            seg = onehot[k * CSB:(k + 1) * CSB]
            within = jnp.dot(tr, seg, preferred_element_type=jnp.float32)
            parts.append(within + carry)
            carry = carry + within[CSB - 1:CSB, :]
        csum = jnp.concatenate(parts, axis=0)
        offs = jnp.dot(carry, up, preferred_element_type=jnp.float32)
        rank = jnp.sum(onehot * (csum - 1.0 + offs), axis=-1, keepdims=True)
        didx = rank.astype(jnp.int32) + (n * BH + bh) * S
        (d0_ref if n == 0 else d1_ref)[0] = didx
        (off0_ref if n == 0 else off1_ref)[0] = offs.astype(jnp.int32)


def _rank(qkv_f, r2):
    return pl.pallas_call(
        _rank_body,
        grid=(B * H,),
        in_specs=[
            pl.BlockSpec((S, 2 * DH), lambda i: (i // H, i % H)),
            pl.BlockSpec((DH, N_HASHES * N_BUCKETS), lambda i: (0, 0)),
        ],
        out_specs=[
            pl.BlockSpec((1, S, 1), lambda i: (i, 0, 0)),
            pl.BlockSpec((1, S, 1), lambda i: (i, 0, 0)),
            pl.BlockSpec((1, 1, N_BUCKETS), lambda i: (i, 0, 0)),
            pl.BlockSpec((1, 1, N_BUCKETS), lambda i: (i, 0, 0)),
        ],
        out_shape=[
            jax.ShapeDtypeStruct((BH, S, 1), jnp.int32),
            jax.ShapeDtypeStruct((BH, S, 1), jnp.int32),
            jax.ShapeDtypeStruct((BH, 1, N_BUCKETS), jnp.int32),
            jax.ShapeDtypeStruct((BH, 1, N_BUCKETS), jnp.int32),
        ],
    )(qkv_f, r2)


def _att_body(sqkv_ref, off_ref, ol_ref):
    qk = sqkv_ref[0, :, 0:DH]
    vv = sqkv_ref[0, :, DH:2 * DH]
    nrm = jnp.sqrt(jnp.sum(qk * qk, axis=-1, keepdims=True)) + 1e-6
    kn = qk / nrm
    k_ext = jnp.concatenate([kn[S - CHUNK:], kn], axis=0)
    v_ext = jnp.concatenate([vv[S - CHUNK:], vv], axis=0)

    offs = off_ref[0]
    pos = lax.broadcasted_iota(jnp.int32, (S, N_BUCKETS), 0)
    bt = jnp.sum((pos >= offs).astype(jnp.int32), axis=-1,
                 keepdims=True)
    b_row = bt.reshape(1, S)
    b_ext = jnp.concatenate([b_row[:, S - CHUNK:], b_row], axis=1)

    qi = lax.broadcasted_iota(jnp.int32, (QB, KB), 0) // CHUNK
    kj = lax.broadcasted_iota(jnp.int32, (QB, KB), 1) // CHUNK
    band = (kj == qi) | (kj == qi + 1)

    for t in range(S // QB):
        p = t * QB
        q = qk[p:p + QB]
        kb = k_ext[p:p + KB]
        vb = v_ext[p:p + KB]
        dots = lax.dot_general(q, kb, (((1,), (1,)), ((), ())),
                               preferred_element_type=jnp.float32) * (1.0 / 8.0)
        mask = band & (bt[p:p + QB] == b_ext[:, p:p + KB])
        dots = jnp.where(mask, dots, -1e9)
        m = jnp.max(dots, axis=-1, keepdims=True)
        e = jnp.exp(dots - m)
        ssum = jnp.sum(e, axis=-1, keepdims=True)
        ol_ref[0, p:p + QB, 0:DH] = jnp.dot(
            e, vb, preferred_element_type=jnp.float32) / ssum
        ol_ref[0, p:p + QB, DH:2 * DH] = jnp.broadcast_to(
            m + jnp.log(ssum), (QB, DH))


def _attention(sqkv, offs):
    return pl.pallas_call(
        _att_body,
        grid=(R,),
        in_specs=[
            pl.BlockSpec((1, S, 2 * DH), lambda i: (i, 0, 0)),
            pl.BlockSpec((1, 1, N_BUCKETS), lambda i: (i, 0, 0)),
        ],
        out_specs=pl.BlockSpec((1, S, 2 * DH), lambda i: (i, 0, 0)),
        out_shape=jax.ShapeDtypeStruct((R, S, 2 * DH), jnp.float32),
    )(sqkv.reshape(R, S, 2 * DH), offs)


NW = 32
SCC = 512
NIDX = SCC // 128


def _sc_sort(qkv4, didx):
    @functools.partial(
        pl.kernel,
        mesh=plsc.VectorSubcoreMesh(core_axis_name="c", subcore_axis_name="s"),
        out_type=jax.ShapeDtypeStruct((R * S, 2 * DH), jnp.float32),
        scratch_types=[
            pltpu.VMEM((S // 128, 128), jnp.int32),
            pltpu.VMEM((SCC, 2 * DH), jnp.float32),
            pltpu.SemaphoreType.DMA,
        ],
    )
    def k(qkv_hbm, didx_hbm, dst_hbm, idx_v, stage_v, sem):
        wid = lax.axis_index("s") * 2 + lax.axis_index("c")
        for t in range(R // NW):
            r = wid + t * NW
            bh = r % BH
            b = bh // H
            h = bh % H
            pltpu.sync_copy(didx_hbm.at[r], idx_v)
            for c in range(S // SCC):
                pltpu.sync_copy(qkv_hbm.at[b, pl.ds(c * SCC, SCC), h], stage_v)
                hs = [pltpu.async_copy(
                    stage_v.at[pl.ds(j * 128, 128)],
                    dst_hbm.at[idx_v.at[c * NIDX + j]], sem)
                    for j in range(NIDX)]
                for hc in hs:
                    hc.wait()

    return k(qkv4, didx)


def _sc_unsort(ol_s, didx):
    @functools.partial(
        pl.kernel,
        mesh=plsc.VectorSubcoreMesh(core_axis_name="c", subcore_axis_name="s"),
        out_type=jax.ShapeDtypeStruct((N_HASHES, B, S, H, 2 * DH), jnp.float32),
        scratch_types=[
            pltpu.VMEM((S // 128, 128), jnp.int32),
            pltpu.VMEM((SCC, 2 * DH), jnp.float32),
            pltpu.SemaphoreType.DMA,
        ],
    )
    def k(ol_hbm, didx_hbm, ou_hbm, idx_v, stage_v, sem):
        wid = lax.axis_index("s") * 2 + lax.axis_index("c")
        for t in range(R // NW):
            r = wid + t * NW
            n = r // BH
            bh = r % BH
            b = bh // H
            h = bh % H
            pltpu.sync_copy(didx_hbm.at[r], idx_v)
            for c in range(S // SCC):
                hs = [pltpu.async_copy(
                    ol_hbm.at[idx_v.at[c * NIDX + j]],
                    stage_v.at[pl.ds(j * 128, 128)], sem)
                    for j in range(NIDX)]
                for hc in hs:
                    hc.wait()
                pltpu.sync_copy(stage_v,
                                ou_hbm.at[n, b, pl.ds(c * SCC, SCC), h])

    return k(ol_s, didx)


FBS = 128


def _ffn_body(x_ref, ou0_ref, ou1_ref, wo_ref, s_ref, b_ref,
              w1_ref, b1_ref, w2_ref, b2_ref, y_ref):
    c0 = ou0_ref[...].reshape(FBS, H, 2 * DH)
    c1 = ou1_ref[...].reshape(FBS, H, 2 * DH)
    d = c0[:, :, DH:DH + 1] - c1[:, :, DH:DH + 1]
    w0 = 1.0 / (1.0 + jnp.exp(-d))
    att = (c1 + w0 * (c0 - c1)).reshape(FBS, 2 * QKV)
    x1 = x_ref[...] + jnp.dot(att, wo_ref[...],
                              preferred_element_type=jnp.float32)
    h2 = _ln(x1, s_ref[...], b_ref[...])
    g = jax.nn.gelu(jnp.dot(h2, w1_ref[...],
                            preferred_element_type=jnp.float32) + b1_ref[...])
    y_ref[...] = x1 + jnp.dot(g, w2_ref[...],
                              preferred_element_type=jnp.float32) + b2_ref[...]


def _ffn(x, ou0, ou1, wo_pad, s, b, w1, b1, w2, b2):
    n = (B * S) // FBS
    return pl.pallas_call(
        _ffn_body,
        grid=(n,),
        in_specs=[
            pl.BlockSpec((FBS, EMB), lambda i: (i, 0)),
            pl.BlockSpec((FBS, 2 * QKV), lambda i: (i, 0)),
            pl.BlockSpec((FBS, 2 * QKV), lambda i: (i, 0)),
            pl.BlockSpec((2 * QKV, EMB), lambda i: (0, 0)),
            pl.BlockSpec((1, EMB), lambda i: (0, 0)),
            pl.BlockSpec((1, EMB), lambda i: (0, 0)),
            pl.BlockSpec((EMB, MLP), lambda i: (0, 0)),
            pl.BlockSpec((1, MLP), lambda i: (0, 0)),
            pl.BlockSpec((MLP, EMB), lambda i: (0, 0)),
            pl.BlockSpec((1, EMB), lambda i: (0, 0)),
        ],
        out_specs=pl.BlockSpec((FBS, EMB), lambda i: (i, 0)),
        out_shape=jax.ShapeDtypeStruct((B * S, EMB), jnp.float32),
    )(x, ou0, ou1, wo_pad, s.reshape(1, EMB), b.reshape(1, EMB), w1,
      b1.reshape(1, MLP), w2, b2.reshape(1, EMB))


def _lnf_body(x_ref, s_ref, b_ref, y_ref):
    y_ref[...] = _ln(x_ref[...], s_ref[...], b_ref[...])


def _lnf(x, s, b):
    n = (B * S) // BS
    return pl.pallas_call(
        _lnf_body,
        grid=(n,),
        in_specs=[
            pl.BlockSpec((BS, EMB), lambda i: (i, 0)),
            pl.BlockSpec((1, EMB), lambda i: (0, 0)),
            pl.BlockSpec((1, EMB), lambda i: (0, 0)),
        ],
        out_specs=pl.BlockSpec((BS, EMB), lambda i: (i, 0)),
        out_shape=jax.ShapeDtypeStruct((B * S, EMB), jnp.float32),
    )(x, s.reshape(1, EMB), b.reshape(1, EMB))


def kernel(inputs, embed, ln1_s, ln1_b, Wqk, Wv, Wo, ln2_s, ln2_b, W1, b1,
           W2, b2, lnf_s, lnf_b):
    pe = _sinusoidal_pe()
    x = jnp.take(embed, inputs.astype(jnp.int32), axis=0).reshape(B * S, EMB)
    x = x + jnp.tile(pe, (B, 1))

    for l in range(L):
        w_int = jnp.concatenate(
            [Wqk[l].reshape(EMB, H, DH), Wv[l].reshape(EMB, H, DH)],
            axis=-1).reshape(EMB, 2 * QKV)
        wo_pad = jnp.concatenate(
            [Wo[l].reshape(H, DH, EMB), jnp.zeros((H, DH, EMB), jnp.float32)],
            axis=1).reshape(2 * QKV, EMB)

        qkv_f = _qkv(x, ln1_s[l], ln1_b[l], w_int)
        qkv4 = qkv_f.reshape(B, S, H, 2 * DH)

        rot = jax.random.normal(jax.random.fold_in(jax.random.key(42), l),
                                (DH, N_HASHES, N_BUCKETS // 2), jnp.float32)
        r2 = jnp.concatenate([rot, -rot], axis=-1).reshape(DH,
                                                           N_HASHES * N_BUCKETS)

        d0, d1, off0, off1 = _rank(qkv_f, r2)
        didx = jnp.stack([d0[:, :, 0], d1[:, :, 0]], 0).reshape(R, S // 128, 128)
        offs = jnp.stack([off0, off1], 0).reshape(R, 1, N_BUCKETS)

        sqkv = _sc_sort(qkv4, didx)
        ol_s = _attention(sqkv, offs)
        ou = _sc_unsort(ol_s.reshape(R * S, 2 * DH), didx)

        x = _ffn(x,
                 ou[0].reshape(B * S, 2 * QKV), ou[1].reshape(B * S, 2 * QKV),
                 wo_pad, ln2_s[l], ln2_b[l], W1[l], b1[l], W2[l], b2[l])

    return _lnf(x, lnf_s, lnf_b).reshape(B, S, EMB)


# speedup vs baseline: 4.7117x; 1.1321x over previous
import functools

import numpy as np
import jax
import jax.numpy as jnp
from jax import lax
from jax.experimental import pallas as pl
from jax.experimental.pallas import tpu as pltpu
from jax.experimental.pallas import tpu_sc as plsc

B, S = 2, 2048
VOCAB, EMB, H, L = 32000, 1024, 16, 2
QKV, MLP = 1024, 4096
DH = QKV // H
N_HASHES, N_BUCKETS, CHUNK = 2, 32, 64
NCH = S // CHUNK
BH = B * H
R = BH * N_HASHES

BS = 512
QB = 256
KB = QB + CHUNK


def _sinusoidal_pe():
    pos = np.arange(S)[:, None].astype(np.float32)
    div = np.exp(np.arange(0, EMB, 2).astype(np.float32) * (-np.log(10000.0) / EMB))
    pe = np.zeros((S, EMB), np.float32)
    pe[:, 0::2] = np.sin(pos * div)
    pe[:, 1::2] = np.cos(pos * div)
    return jnp.asarray(pe)


def _ln(x, s, b):
    m = jnp.mean(x, axis=-1, keepdims=True)
    v = jnp.mean(jnp.square(x - m), axis=-1, keepdims=True)
    return (x - m) * lax.rsqrt(v + 1e-6) * s + b


def _qkv_body(x_ref, s_ref, b_ref, w_ref, qkv_ref):
    h = _ln(x_ref[...], s_ref[...], b_ref[...])
    qkv_ref[...] = jnp.dot(h, w_ref[...], preferred_element_type=jnp.float32)


def _qkv(x, s, b, w_int):
    n = (B * S) // BS
    return pl.pallas_call(
        _qkv_body,
        grid=(n,),
        in_specs=[
            pl.BlockSpec((BS, EMB), lambda i: (i, 0)),
            pl.BlockSpec((1, EMB), lambda i: (0, 0)),
            pl.BlockSpec((1, EMB), lambda i: (0, 0)),
            pl.BlockSpec((EMB, 2 * QKV), lambda i: (0, 0)),
        ],
        out_specs=pl.BlockSpec((BS, 2 * QKV), lambda i: (i, 0)),
        out_shape=jax.ShapeDtypeStruct((B * S, 2 * QKV), jnp.float32),
    )(x, s.reshape(1, EMB), b.reshape(1, EMB), w_int)


CSB = 256


def _rank_body(qk_ref, r2_ref, d0_ref, d1_ref, off0_ref, off1_ref):
    i = pl.program_id(0)

    lane = lax.broadcasted_iota(jnp.int32, (S, N_BUCKETS), 1)
    tr = (lax.broadcasted_iota(jnp.int32, (CSB, CSB), 0)
          >= lax.broadcasted_iota(jnp.int32, (CSB, CSB), 1)).astype(jnp.float32)
    up = (lax.broadcasted_iota(jnp.int32, (N_BUCKETS, N_BUCKETS), 0)
          < lax.broadcasted_iota(jnp.int32, (N_BUCKETS, N_BUCKETS), 1)
          ).astype(jnp.float32)

    bh = i
    qk = qk_ref[:, 0:DH]
    scores = jnp.dot(qk, r2_ref[...], preferred_element_type=jnp.float32)
    for n in range(N_HASHES):
        sn = scores[:, n * N_BUCKETS:(n + 1) * N_BUCKETS]
        m = jnp.max(sn, axis=-1, keepdims=True)
        bidx = jnp.min(jnp.where(sn == m, lane, N_BUCKETS), axis=-1,
                       keepdims=True)
        onehot = (lane == bidx).astype(jnp.float32)
        carry = jnp.zeros((1, N_BUCKETS), jnp.float32)
        parts = []
        for k in range(S // CSB):
            seg = onehot[k * CSB:(k + 1) * CSB]
            within = jnp.dot(tr, seg, preferred_element_type=jnp.float32)
            parts.append(within + carry)
            carry = carry + within[CSB - 1:CSB, :]
        csum = jnp.concatenate(parts, axis=0)
        offs = jnp.dot(carry, up, preferred_element_type=jnp.float32)
        rank = jnp.sum(onehot * (csum - 1.0 + offs), axis=-1, keepdims=True)
        didx = rank.astype(jnp.int32) + bh * S
        (d0_ref if n == 0 else d1_ref)[0] = didx
        (off0_ref if n == 0 else off1_ref)[0] = offs.astype(jnp.int32)


def _rank(qkv_f, r2):
    return pl.pallas_call(
        _rank_body,
        grid=(B * H,),
        in_specs=[
            pl.BlockSpec((S, 2 * DH), lambda i: (i // H, i % H)),
            pl.BlockSpec((DH, N_HASHES * N_BUCKETS), lambda i: (0, 0)),
        ],
        out_specs=[
            pl.BlockSpec((1, S, 1), lambda i: (i, 0, 0)),
            pl.BlockSpec((1, S, 1), lambda i: (i, 0, 0)),
            pl.BlockSpec((1, 1, N_BUCKETS), lambda i: (i, 0, 0)),
            pl.BlockSpec((1, 1, N_BUCKETS), lambda i: (i, 0, 0)),
        ],
        out_shape=[
            jax.ShapeDtypeStruct((BH, S, 1), jnp.int32),
            jax.ShapeDtypeStruct((BH, S, 1), jnp.int32),
            jax.ShapeDtypeStruct((BH, 1, N_BUCKETS), jnp.int32),
            jax.ShapeDtypeStruct((BH, 1, N_BUCKETS), jnp.int32),
        ],
    )(qkv_f, r2)


def _att_body(sqkv_ref, off_ref, ol_ref):
    qk = sqkv_ref[0, :, 0:DH]
    vv = sqkv_ref[0, :, DH:2 * DH]
    nrm = jnp.sqrt(jnp.sum(qk * qk, axis=-1, keepdims=True)) + 1e-6
    kn = qk / nrm
    k_ext = jnp.concatenate([kn[S - CHUNK:], kn], axis=0)
    v_ext = jnp.concatenate([vv[S - CHUNK:], vv], axis=0)

    offs = off_ref[0]
    pos = lax.broadcasted_iota(jnp.int32, (S, N_BUCKETS), 0)
    bt = jnp.sum((pos >= offs).astype(jnp.int32), axis=-1,
                 keepdims=True)
    b_row = bt.reshape(1, S)
    b_ext = jnp.concatenate([b_row[:, S - CHUNK:], b_row], axis=1)

    qi = lax.broadcasted_iota(jnp.int32, (QB, KB), 0) // CHUNK
    kj = lax.broadcasted_iota(jnp.int32, (QB, KB), 1) // CHUNK
    band = (kj == qi) | (kj == qi + 1)

    for t in range(S // QB):
        p = t * QB
        q = qk[p:p + QB]
        kb = k_ext[p:p + KB]
        vb = v_ext[p:p + KB]
        dots = lax.dot_general(q, kb, (((1,), (1,)), ((), ())),
                               preferred_element_type=jnp.float32) * (1.0 / 8.0)
        mask = band & (bt[p:p + QB] == b_ext[:, p:p + KB])
        dots = jnp.where(mask, dots, -1e9)
        m = jnp.max(dots, axis=-1, keepdims=True)
        e = jnp.exp(dots - m)
        ssum = jnp.sum(e, axis=-1, keepdims=True)
        ol_ref[0, p:p + QB, 0:DH] = jnp.dot(
            e, vb, preferred_element_type=jnp.float32) / ssum
        ol_ref[0, p:p + QB, DH:2 * DH] = jnp.broadcast_to(
            m + jnp.log(ssum), (QB, DH))


def _attention(sqkv, offs):
    return pl.pallas_call(
        _att_body,
        grid=(BH,),
        in_specs=[
            pl.BlockSpec((1, S, 2 * DH), lambda i: (i, 0, 0)),
            pl.BlockSpec((1, 1, N_BUCKETS), lambda i: (i, 0, 0)),
        ],
        out_specs=pl.BlockSpec((1, S, 2 * DH), lambda i: (i, 0, 0)),
        out_shape=jax.ShapeDtypeStruct((BH, S, 2 * DH), jnp.float32),
    )(sqkv.reshape(BH, S, 2 * DH), offs)


NW = 32
SCC = 512
NIDX = SCC // 128


def _sc_sort(qkv4, didx):
    @functools.partial(
        pl.kernel,
        mesh=plsc.VectorSubcoreMesh(core_axis_name="c", subcore_axis_name="s"),
        out_type=jax.ShapeDtypeStruct((BH * S, 2 * DH), jnp.float32),
        scratch_types=[
            pltpu.VMEM((S // 128, 128), jnp.int32),
            pltpu.VMEM((SCC, 2 * DH), jnp.float32),
            pltpu.SemaphoreType.DMA,
        ],
    )
    def k(qkv_hbm, didx_hbm, dst_hbm, idx_v, stage_v, sem):
        bh = lax.axis_index("s") * 2 + lax.axis_index("c")
        b = bh // H
        h = bh % H
        pltpu.sync_copy(didx_hbm.at[bh], idx_v)
        for c in range(S // SCC):
            pltpu.sync_copy(qkv_hbm.at[b, pl.ds(c * SCC, SCC), h], stage_v)
            hs = [pltpu.async_copy(
                stage_v.at[pl.ds(j * 128, 128)],
                dst_hbm.at[idx_v.at[c * NIDX + j]], sem)
                for j in range(NIDX)]
            for hc in hs:
                hc.wait()

    return k(qkv4, didx)


def _sc_unsort(ol_s, didx):
    @functools.partial(
        pl.kernel,
        mesh=plsc.VectorSubcoreMesh(core_axis_name="c", subcore_axis_name="s"),
        out_type=jax.ShapeDtypeStruct((B, S, H, 2 * DH), jnp.float32),
        scratch_types=[
            pltpu.VMEM((S // 128, 128), jnp.int32),
            pltpu.VMEM((SCC, 2 * DH), jnp.float32),
            pltpu.SemaphoreType.DMA,
        ],
    )
    def k(ol_hbm, didx_hbm, ou_hbm, idx_v, stage_v, sem):
        bh = lax.axis_index("s") * 2 + lax.axis_index("c")
        b = bh // H
        h = bh % H
        pltpu.sync_copy(didx_hbm.at[bh], idx_v)
        for c in range(S // SCC):
            hs = [pltpu.async_copy(
                ol_hbm.at[idx_v.at[c * NIDX + j]],
                stage_v.at[pl.ds(j * 128, 128)], sem)
                for j in range(NIDX)]
            for hc in hs:
                hc.wait()
            pltpu.sync_copy(stage_v,
                            ou_hbm.at[b, pl.ds(c * SCC, SCC), h])

    return k(ol_s, didx)


FBS = 128


def _ffn_body(x_ref, ou0_ref, ou1_ref, wo_ref, s_ref, b_ref,
              w1_ref, b1_ref, w2_ref, b2_ref, y_ref):
    c0 = ou0_ref[...].reshape(FBS, H, 2 * DH)
    c1 = ou1_ref[...].reshape(FBS, H, 2 * DH)
    d = c0[:, :, DH:DH + 1] - c1[:, :, DH:DH + 1]
    w0 = 1.0 / (1.0 + jnp.exp(-d))
    att = (c1 + w0 * (c0 - c1)).reshape(FBS, 2 * QKV)
    x1 = x_ref[...] + jnp.dot(att, wo_ref[...],
                              preferred_element_type=jnp.float32)
    h2 = _ln(x1, s_ref[...], b_ref[...])
    g = jax.nn.gelu(jnp.dot(h2, w1_ref[...],
                            preferred_element_type=jnp.float32) + b1_ref[...])
    y_ref[...] = x1 + jnp.dot(g, w2_ref[...],
                              preferred_element_type=jnp.float32) + b2_ref[...]


def _ffn(x, ou0, ou1, wo_pad, s, b, w1, b1, w2, b2):
    n = (B * S) // FBS
    return pl.pallas_call(
        _ffn_body,
        grid=(n,),
        in_specs=[
            pl.BlockSpec((FBS, EMB), lambda i: (i, 0)),
            pl.BlockSpec((FBS, 2 * QKV), lambda i: (i, 0)),
            pl.BlockSpec((FBS, 2 * QKV), lambda i: (i, 0)),
            pl.BlockSpec((2 * QKV, EMB), lambda i: (0, 0)),
            pl.BlockSpec((1, EMB), lambda i: (0, 0)),
            pl.BlockSpec((1, EMB), lambda i: (0, 0)),
            pl.BlockSpec((EMB, MLP), lambda i: (0, 0)),
            pl.BlockSpec((1, MLP), lambda i: (0, 0)),
            pl.BlockSpec((MLP, EMB), lambda i: (0, 0)),
            pl.BlockSpec((1, EMB), lambda i: (0, 0)),
        ],
        out_specs=pl.BlockSpec((FBS, EMB), lambda i: (i, 0)),
        out_shape=jax.ShapeDtypeStruct((B * S, EMB), jnp.float32),
    )(x, ou0, ou1, wo_pad, s.reshape(1, EMB), b.reshape(1, EMB), w1,
      b1.reshape(1, MLP), w2, b2.reshape(1, EMB))


def _lnf_body(x_ref, s_ref, b_ref, y_ref):
    y_ref[...] = _ln(x_ref[...], s_ref[...], b_ref[...])


def _lnf(x, s, b):
    n = (B * S) // BS
    return pl.pallas_call(
        _lnf_body,
        grid=(n,),
        in_specs=[
            pl.BlockSpec((BS, EMB), lambda i: (i, 0)),
            pl.BlockSpec((1, EMB), lambda i: (0, 0)),
            pl.BlockSpec((1, EMB), lambda i: (0, 0)),
        ],
        out_specs=pl.BlockSpec((BS, EMB), lambda i: (i, 0)),
        out_shape=jax.ShapeDtypeStruct((B * S, EMB), jnp.float32),
    )(x, s.reshape(1, EMB), b.reshape(1, EMB))


def kernel(inputs, embed, ln1_s, ln1_b, Wqk, Wv, Wo, ln2_s, ln2_b, W1, b1,
           W2, b2, lnf_s, lnf_b):
    pe = _sinusoidal_pe()
    x = jnp.take(embed, inputs.astype(jnp.int32), axis=0).reshape(B * S, EMB)
    x = x + jnp.tile(pe, (B, 1))

    for l in range(L):
        w_int = jnp.concatenate(
            [Wqk[l].reshape(EMB, H, DH), Wv[l].reshape(EMB, H, DH)],
            axis=-1).reshape(EMB, 2 * QKV)
        wo_pad = jnp.concatenate(
            [Wo[l].reshape(H, DH, EMB), jnp.zeros((H, DH, EMB), jnp.float32)],
            axis=1).reshape(2 * QKV, EMB)

        qkv_f = _qkv(x, ln1_s[l], ln1_b[l], w_int)
        qkv4 = qkv_f.reshape(B, S, H, 2 * DH)

        rot = jax.random.normal(jax.random.fold_in(jax.random.key(42), l),
                                (DH, N_HASHES, N_BUCKETS // 2), jnp.float32)
        r2 = jnp.concatenate([rot, -rot], axis=-1).reshape(DH,
                                                           N_HASHES * N_BUCKETS)

        d0, d1, off0, off1 = _rank(qkv_f, r2)
        ou = []
        for dn, offn in ((d0, off0), (d1, off1)):
            didx = dn.reshape(BH, S // 128, 128)
            sqkv = _sc_sort(qkv4, didx)
            ol_s = _attention(sqkv, offn.reshape(BH, 1, N_BUCKETS))
            ou.append(_sc_unsort(ol_s.reshape(BH * S, 2 * DH), didx))

        x = _ffn(x,
                 ou[0].reshape(B * S, 2 * QKV), ou[1].reshape(B * S, 2 * QKV),
                 wo_pad, ln2_s[l], ln2_b[l], W1[l], b1[l], W2[l], b2[l])

    return _lnf(x, lnf_s, lnf_b).reshape(B, S, EMB)
